# baseline (speedup 1.0000x reference)
import functools

import jax
import jax.numpy as jnp
from jax import lax
from jax.experimental import pallas as pl
from jax.experimental.pallas import tpu as pltpu
from jax.experimental.pallas import tpu_sc as plsc

N = 50000
E = 40000
D = 128
NUM_REL = 14
NB = 1000
GRID = N // NB
NBD = 1000
GRIDD = N // NBD
B_E = 128
N_TILES = 16
EPAD = 40960
NBATCH = EPAD // (N_TILES * B_E)
NPLANE = 8
NROUND = NPLANE // 2
SEG = EPAD // N_TILES
CAPB = SEG // B_E + 14
CAP = CAPB * B_E
Q = 7000
ZROWS = 456
SP_ROWS = N_TILES * ZROWS
WB_TILES = Q // ZROWS
WB_REM = Q - WB_TILES * ZROWS
ZCHUNK = 128


def _gn_rows(x, w, b):
    mu = jnp.mean(x, axis=1, keepdims=True)
    xc = x - mu
    var = jnp.mean(xc * xc, axis=1, keepdims=True)
    return xc * lax.rsqrt(var + 1e-5) * w + b


def _stage_a_body(nodes, iw1, ib1, iw2, ignw, ignb, sw1, sb1, sw2, sgnw,
                  sgnb, mw, mgnw, mgnb, wcat, feat_out, y_out):
    x = nodes[...]
    f = jnp.maximum(x[:, 0:1] * iw1[0:1, :] + x[:, 1:2] * iw1[1:2, :]
                    + ib1[...], 0.0)
    f = _gn_rows(jnp.dot(f, iw2[...], preferred_element_type=jnp.float32),
                 ignw[...], ignb[...])
    s = jnp.maximum(x[:, 2:3] * sw1[0:1, :] + x[:, 3:4] * sw1[1:2, :]
                    + sb1[...], 0.0)
    s = _gn_rows(jnp.dot(s, sw2[...], preferred_element_type=jnp.float32),
                 sgnw[...], sgnb[...])
    f = jnp.maximum(f + s, 0.0)
    m = jnp.dot(f, mw[0:D, :], preferred_element_type=jnp.float32)
    m = (m + x[:, 4:5] * mw[D:D + 1, :] + x[:, 5:6] * mw[D + 1:D + 2, :]
         + x[:, 6:7] * mw[D + 2:D + 3, :] + x[:, 7:8] * mw[D + 3:D + 4, :])
    f = jnp.maximum(_gn_rows(m, mgnw[...], mgnb[...]), 0.0)
    feat_out[...] = f
    y_big = jnp.dot(f, wcat[...], preferred_element_type=jnp.float32)
    for k in range(NUM_REL + 1):
        y_out[k] = y_big[:, k * D:(k + 1) * D]


def _stage_a(nodes, pv):
    full = lambda shp: pl.BlockSpec(shp, lambda i: tuple(0 for _ in shp))
    return pl.pallas_call(
        _stage_a_body,
        grid=(GRID,),
        in_specs=[
            pl.BlockSpec((NB, 8), lambda i: (i, 0)),
            full((2, D)), full((1, D)), full((D, D)), full((1, D)),
            full((1, D)),
            full((2, D)), full((1, D)), full((D, D)), full((1, D)),
            full((1, D)),
            full((D + 4, D)), full((1, D)), full((1, D)),
            full((D, (NUM_REL + 1) * D)),
        ],
        out_specs=[
            pl.BlockSpec((NB, D), lambda i: (i, 0)),
            pl.BlockSpec((NUM_REL + 1, NB, D), lambda i: (0, i, 0)),
        ],
        out_shape=[
            jax.ShapeDtypeStruct((N, D), jnp.float32),
            jax.ShapeDtypeStruct((NUM_REL + 1, N, D), jnp.float32),
        ],
    )(nodes, *pv)


def _stage_d_feat(t, y, res, nw, nb_, c2w, c2gw, c2gb):
    temp = t[0] + y[0]
    x1 = jnp.maximum(_gn_rows(temp, nw[...], nb_[...]), 0.0)
    z = _gn_rows(jnp.dot(x1, c2w[...], preferred_element_type=jnp.float32),
                 c2gw[...], c2gb[...])
    return jnp.maximum(z + res[...], 0.0)


def _stage_d_body(t, y, res, nw, nb_, c2w, c2gw, c2gb, wcat, feat_out,
                  y_out):
    f = _stage_d_feat(t, y, res, nw, nb_, c2w, c2gw, c2gb)
    feat_out[...] = f
    y_big = jnp.dot(f, wcat[...], preferred_element_type=jnp.float32)
    for k in range(NUM_REL + 1):
        y_out[k] = y_big[:, k * D:(k + 1) * D]


def _stage_d_body_last(t, y, res, nw, nb_, c2w, c2gw, c2gb, feat_out):
    feat_out[...] = _stage_d_feat(t, y, res, nw, nb_, c2w, c2gw, c2gb)


_BPP = Q // NBD


def _stage_d(temp4, y, res, nw, nb_, c2w, c2gw, c2gb, wcat):
    last = wcat is None
    full = lambda shp: pl.BlockSpec(shp, lambda i: tuple(0 for _ in shp))
    body = _stage_d_body_last if last else _stage_d_body
    in_specs = [
        pl.BlockSpec((1, NBD, D), lambda i: (i // _BPP, i % _BPP, 0)),
        pl.BlockSpec((1, NBD, D), lambda i: (0, i, 0)),
        pl.BlockSpec((NBD, D), lambda i: (i, 0)),
        full((1, D)), full((1, D)), full((D, D)), full((1, D)),
        full((1, D)),
    ]
    out_specs = [pl.BlockSpec((NBD, D), lambda i: (i, 0))]
    out_shape = [jax.ShapeDtypeStruct((N, D), jnp.float32)]
    args = [temp4, y, res, nw, nb_, c2w, c2gw, c2gb]
    if not last:
        in_specs.append(full((D, (NUM_REL + 1) * D)))
        out_specs.append(pl.BlockSpec((NUM_REL + 1, NBD, D),
                                      lambda i: (0, i, 0)))
        out_shape.append(
            jax.ShapeDtypeStruct((NUM_REL + 1, N, D), jnp.float32))
        args.append(wcat)
    res_ = pl.pallas_call(
        body, grid=(GRIDD,), in_specs=in_specs, out_specs=out_specs,
        out_shape=out_shape,
    )(*args)
    return res_ if not last else (res_[0], None)


def _make_sc_scatter():
    return functools.partial(
        pl.kernel,
        mesh=plsc.VectorSubcoreMesh(core_axis_name="c",
                                    subcore_axis_name="s", num_cores=2),
        out_type=jax.ShapeDtypeStruct((NPLANE, SP_ROWS, D), jnp.float32),
        scratch_types=[
            pltpu.VMEM((CAPB, B_E), jnp.int32),
            pltpu.VMEM((CAPB, B_E), jnp.int32),
            pltpu.VMEM((N_TILES, 32), jnp.int32),
            pltpu.VMEM((B_E, D), jnp.float32),
            pltpu.VMEM((B_E, D), jnp.float32),
            pltpu.VMEM((ZCHUNK, D), jnp.float32),
            pltpu.VMEM_SHARED((SP_ROWS, D), jnp.float32),
            pltpu.SemaphoreType.DMA,
            pltpu.SemaphoreType.DMA,
        ],
    )(_sc_scatter_body)


def _sc_scatter_body(y_hbm, src_hbm, dst_hbm, nbb_hbm, out_hbm, src_v, dst_v,
                nb_v, rows0_v, rows1_v, zero_v, acc_sh, sem0, sem1):
    cid = lax.axis_index("c")
    sid = lax.axis_index("s")

    zv = jnp.zeros((16,), jnp.float32)
    lane = lax.iota(jnp.int32, 16)
    z16 = lane * 0

    def zfill(k, carry):
        for c in range(D // 16):
            zero_v[k, 16 * c:16 * (c + 1)] = zv
        return carry

    lax.fori_loop(0, ZCHUNK, zfill, 0)

    zbase = sid * ZROWS

    for r in range(NROUND):
        q = cid + 2 * r

        def zcopy(k, carry):
            pltpu.sync_copy(zero_v, acc_sh.at[pl.ds(zbase + k * ZCHUNK,
                                                    ZCHUNK)])
            return carry

        lax.fori_loop(0, ZROWS // ZCHUNK, zcopy, 0)
        rem = ZROWS % ZCHUNK
        pltpu.sync_copy(zero_v.at[pl.ds(0, rem)],
                        acc_sh.at[pl.ds(zbase + (ZROWS // ZCHUNK) * ZCHUNK,
                                        rem)])
        plsc.subcore_barrier()

        def rel_body(j, carry):
            pltpu.sync_copy(src_hbm.at[j, sid], src_v)
            pltpu.sync_copy(dst_hbm.at[j, sid], dst_v)
            pltpu.sync_copy(nbb_hbm.at[j], nb_v)
            bq = nb_v[sid, pl.ds(q, 16)]
            plo = bq[0]
            phi = bq[1]

            @pl.when(phi > plo)
            def _pipe():
                pltpu.async_copy(y_hbm.at[src_v.at[2 * plo]], rows0_v, sem0)

                def pair_body(h, c2):
                    b0 = 2 * h
                    b1 = 2 * h + 1
                    pltpu.make_async_copy(y_hbm.at[src_v.at[b0]], rows0_v,
                                          sem0).wait()
                    pltpu.async_copy(y_hbm.at[src_v.at[b1]], rows1_v, sem1)
                    pltpu.sync_copy(rows0_v, acc_sh.at[dst_v.at[b0]],
                                    add=True)
                    pltpu.make_async_copy(y_hbm.at[src_v.at[b1]], rows1_v,
                                          sem1).wait()
                    bn = jnp.minimum(b0 + 2, 2 * phi - 1)
                    pltpu.async_copy(y_hbm.at[src_v.at[bn]], rows0_v, sem0)
                    pltpu.sync_copy(rows1_v, acc_sh.at[dst_v.at[b1]],
                                    add=True)
                    return c2

                lax.fori_loop(plo, phi, pair_body, 0)
                pltpu.make_async_copy(y_hbm.at[src_v.at[0]], rows0_v,
                                      sem0).wait()

            return carry

        lax.fori_loop(0, NUM_REL, rel_body, 0)
        plsc.subcore_barrier()

        @pl.when(sid < WB_TILES)
        def _wb_full():
            pltpu.sync_copy(acc_sh.at[pl.ds(zbase, ZROWS)],
                            out_hbm.at[q, pl.ds(zbase, ZROWS)])

        @pl.when(sid == WB_TILES)
        def _wb_part():
            pltpu.sync_copy(acc_sh.at[pl.ds(zbase, WB_REM)],
                            out_hbm.at[q, pl.ds(zbase, WB_REM)])

        if r < NROUND - 1:
            plsc.subcore_barrier()


def _prep_indices(indexes):
    idx = indexes.astype(jnp.int32)
    dst = idx[:, 0::2].T
    src = idx[:, 1::2].T
    pad = EPAD - E
    ar = jnp.arange(pad, dtype=jnp.int32)
    pad_src = (ar * 977) % N
    pad_dst = NPLANE * Q + (ar % 960)
    srcp = jnp.concatenate(
        [src, jnp.broadcast_to(pad_src, (NUM_REL, pad))], axis=1)
    dstp = jnp.concatenate(
        [dst, jnp.broadcast_to(pad_dst, (NUM_REL, pad))], axis=1)
    jrel = jnp.arange(NUM_REL, dtype=jnp.int32)[:, None]
    srcf = ((jrel + 1) * N + srcp).reshape(NUM_REL, N_TILES, SEG)
    dstg = dstp.reshape(NUM_REL, N_TILES, SEG)

    plane = dstg // Q
    e_l = jnp.arange(SEG, dtype=jnp.int32)
    key = plane * 4096 + e_l
    skey, ssrc, sdst = lax.sort((key, srcf, dstg), num_keys=1)
    splane = skey // 4096

    pv = jnp.arange(NPLANE + 2, dtype=jnp.int32)
    starts = jnp.sum(splane[:, :, None, :] < pv[None, None, :, None],
                     axis=-1)
    cnts = starts[:, :, 1:] - starts[:, :, :-1]
    prs = (cnts + 2 * B_E - 1) // (2 * B_E)
    pp_start = jnp.concatenate(
        [jnp.zeros_like(prs[:, :, :1]), jnp.cumsum(prs, axis=-1)],
        axis=-1)

    k_sl = jnp.arange(CAP, dtype=jnp.int32)
    kp = k_sl // (2 * B_E)
    pk = jnp.sum(kp[None, None, None, :] >= pp_start[:, :, 1:, None],
                 axis=2)
    pk7 = jnp.minimum(pk, NPLANE)
    pbs = jnp.take_along_axis(pp_start, pk7, axis=-1)
    sbase = jnp.take_along_axis(starts, pk7, axis=-1)
    cpk = jnp.take_along_axis(
        jnp.concatenate([cnts, jnp.zeros_like(cnts[:, :, :1])], axis=-1),
        pk7, axis=-1)
    rank = k_sl[None, None, :] - pbs * (2 * B_E)
    valid = (rank < cpk) & (pk7 < NPLANE) & (pk < NPLANE + 1)
    i_k = jnp.clip(sbase + rank, 0, SEG - 1)
    g_src = jnp.take_along_axis(ssrc, i_k, axis=-1)
    g_dst = jnp.take_along_axis(sdst, i_k, axis=-1) - pk7 * Q
    dum_src = (k_sl * 977) % N
    dum_dst = Q + (k_sl * 37) % (SP_ROWS - Q - 8)
    out_src = jnp.where(valid, g_src, dum_src[None, None, :])
    out_dst = jnp.where(valid, g_dst, dum_dst[None, None, :])
    out_src = out_src.reshape(NUM_REL, N_TILES, CAPB, B_E)
    out_dst = out_dst.reshape(NUM_REL, N_TILES, CAPB, B_E)

    nbb = jnp.concatenate(
        [pp_start,
         jnp.zeros((NUM_REL, N_TILES, 32 - pp_start.shape[-1]), jnp.int32)],
        axis=-1)
    return out_src, out_dst, nbb


def kernel(nodes, params, indexes):
    p = params
    src_flat, dst_flat, nbb = _prep_indices(indexes)
    r1 = lambda a: a.reshape(1, D)
    wcats = [
        jnp.transpose(
            jnp.concatenate([p['ctr_w'][i:i + 1], p['rel_w'][i]], axis=0),
            (1, 0, 2)).reshape(D, (NUM_REL + 1) * D)
        for i in range(4)
    ]
    pv = [
        p['input_w1'], r1(p['input_b1']), p['input_w2'],
        r1(p['input_gn_w']), r1(p['input_gn_b']),
        p['seg_w1'], r1(p['seg_b1']), p['seg_w2'],
        r1(p['seg_gn_w']), r1(p['seg_gn_b']),
        p['meta_w'], r1(p['meta_gn_w']), r1(p['meta_gn_b']),
        wcats[0],
    ]
    feat, y = _stage_a(nodes, pv)
    sc_scatter = _make_sc_scatter()
    for i in range(4):
        y_flat = y.reshape((NUM_REL + 1) * N, D)
        temp4 = sc_scatter(y_flat, src_flat, dst_flat, nbb)
        wcat_next = wcats[i + 1] if i < 3 else None
        feat, y = _stage_d(
            temp4, y, feat,
            r1(p['norm_w'][i]), r1(p['norm_b'][i]),
            p['ctr2_w'][i],
            r1(p['ctr2_gn_w'][i]), r1(p['ctr2_gn_b'][i]),
            wcat_next)
    return (feat, nodes[:, :2])


# speedup vs baseline: 5.2805x; 5.2805x over previous
import functools

import jax
import jax.numpy as jnp
from jax import lax
from jax.experimental import pallas as pl
from jax.experimental.pallas import tpu as pltpu
from jax.experimental.pallas import tpu_sc as plsc

N = 50000
E = 40000
D = 128
NUM_REL = 14
NB = 1000
GRID = N // NB
NBD = 1000
GRIDD = N // NBD
B_E = 128
N_TILES = 16
EPAD = 40960
NBATCH = EPAD // (N_TILES * B_E)
NPLANE = 8
NROUND = NPLANE // 2
SEG = EPAD // N_TILES
CAPB = SEG // B_E + 16
CAP = CAPB * B_E
Q = 7000
ZROWS = 456
SP_ROWS = N_TILES * ZROWS
WB_TILES = Q // ZROWS
WB_REM = Q - WB_TILES * ZROWS
ZCHUNK = 128


def _gn_rows(x, w, b):
    mu = jnp.mean(x, axis=1, keepdims=True)
    xc = x - mu
    var = jnp.mean(xc * xc, axis=1, keepdims=True)
    return xc * lax.rsqrt(var + 1e-5) * w + b


def _stage_a_body(nodes, iw1, ib1, iw2, ignw, ignb, sw1, sb1, sw2, sgnw,
                  sgnb, mw, mgnw, mgnb, wcat, feat_out, y_out):
    x = nodes[...]
    f = jnp.maximum(x[:, 0:1] * iw1[0:1, :] + x[:, 1:2] * iw1[1:2, :]
                    + ib1[...], 0.0)
    f = _gn_rows(jnp.dot(f, iw2[...], preferred_element_type=jnp.float32),
                 ignw[...], ignb[...])
    s = jnp.maximum(x[:, 2:3] * sw1[0:1, :] + x[:, 3:4] * sw1[1:2, :]
                    + sb1[...], 0.0)
    s = _gn_rows(jnp.dot(s, sw2[...], preferred_element_type=jnp.float32),
                 sgnw[...], sgnb[...])
    f = jnp.maximum(f + s, 0.0)
    m = jnp.dot(f, mw[0:D, :], preferred_element_type=jnp.float32)
    m = (m + x[:, 4:5] * mw[D:D + 1, :] + x[:, 5:6] * mw[D + 1:D + 2, :]
         + x[:, 6:7] * mw[D + 2:D + 3, :] + x[:, 7:8] * mw[D + 3:D + 4, :])
    f = jnp.maximum(_gn_rows(m, mgnw[...], mgnb[...]), 0.0)
    feat_out[...] = f
    y_big = jnp.dot(f, wcat[...], preferred_element_type=jnp.float32)
    for k in range(NUM_REL + 1):
        y_out[k] = y_big[:, k * D:(k + 1) * D]


def _stage_a(nodes, pv):
    full = lambda shp: pl.BlockSpec(shp, lambda i: tuple(0 for _ in shp))
    return pl.pallas_call(
        _stage_a_body,
        grid=(GRID,),
        in_specs=[
            pl.BlockSpec((NB, 8), lambda i: (i, 0)),
            full((2, D)), full((1, D)), full((D, D)), full((1, D)),
            full((1, D)),
            full((2, D)), full((1, D)), full((D, D)), full((1, D)),
            full((1, D)),
            full((D + 4, D)), full((1, D)), full((1, D)),
            full((D, (NUM_REL + 1) * D)),
        ],
        out_specs=[
            pl.BlockSpec((NB, D), lambda i: (i, 0)),
            pl.BlockSpec((NUM_REL + 1, NB, D), lambda i: (0, i, 0)),
        ],
        out_shape=[
            jax.ShapeDtypeStruct((N, D), jnp.float32),
            jax.ShapeDtypeStruct((NUM_REL + 1, N, D), jnp.float32),
        ],
    )(nodes, *pv)


def _stage_d_feat(t, y, res, nw, nb_, c2w, c2gw, c2gb):
    temp = t[0] + y[0]
    x1 = jnp.maximum(_gn_rows(temp, nw[...], nb_[...]), 0.0)
    z = _gn_rows(jnp.dot(x1, c2w[...], preferred_element_type=jnp.float32),
                 c2gw[...], c2gb[...])
    return jnp.maximum(z + res[...], 0.0)


def _stage_d_body(t, y, res, nw, nb_, c2w, c2gw, c2gb, wcat, feat_out,
                  y_out):
    f = _stage_d_feat(t, y, res, nw, nb_, c2w, c2gw, c2gb)
    feat_out[...] = f
    y_big = jnp.dot(f, wcat[...], preferred_element_type=jnp.float32)
    for k in range(NUM_REL + 1):
        y_out[k] = y_big[:, k * D:(k + 1) * D]


def _stage_d_body_last(t, y, res, nw, nb_, c2w, c2gw, c2gb, feat_out):
    feat_out[...] = _stage_d_feat(t, y, res, nw, nb_, c2w, c2gw, c2gb)


_BPP = Q // NBD


def _stage_d(temp4, y, res, nw, nb_, c2w, c2gw, c2gb, wcat):
    last = wcat is None
    full = lambda shp: pl.BlockSpec(shp, lambda i: tuple(0 for _ in shp))
    body = _stage_d_body_last if last else _stage_d_body
    in_specs = [
        pl.BlockSpec((1, NBD, D), lambda i: (i // _BPP, i % _BPP, 0)),
        pl.BlockSpec((1, NBD, D), lambda i: (0, i, 0)),
        pl.BlockSpec((NBD, D), lambda i: (i, 0)),
        full((1, D)), full((1, D)), full((D, D)), full((1, D)),
        full((1, D)),
    ]
    out_specs = [pl.BlockSpec((NBD, D), lambda i: (i, 0))]
    out_shape = [jax.ShapeDtypeStruct((N, D), jnp.float32)]
    args = [temp4, y, res, nw, nb_, c2w, c2gw, c2gb]
    if not last:
        in_specs.append(full((D, (NUM_REL + 1) * D)))
        out_specs.append(pl.BlockSpec((NUM_REL + 1, NBD, D),
                                      lambda i: (0, i, 0)))
        out_shape.append(
            jax.ShapeDtypeStruct((NUM_REL + 1, N, D), jnp.float32))
        args.append(wcat)
    res_ = pl.pallas_call(
        body, grid=(GRIDD,), in_specs=in_specs, out_specs=out_specs,
        out_shape=out_shape,
    )(*args)
    return res_ if not last else (res_[0], None)


def _make_sc_scatter():
    return functools.partial(
        pl.kernel,
        mesh=plsc.VectorSubcoreMesh(core_axis_name="c",
                                    subcore_axis_name="s", num_cores=2),
        out_type=jax.ShapeDtypeStruct((NPLANE, SP_ROWS, D), jnp.float32),
        scratch_types=[
            pltpu.VMEM((CAPB, B_E), jnp.int32),
            pltpu.VMEM((CAPB, B_E), jnp.int32),
            pltpu.VMEM((N_TILES, 32), jnp.int32),
            pltpu.VMEM((B_E, D), jnp.float32),
            pltpu.VMEM((B_E, D), jnp.float32),
            pltpu.VMEM((ZCHUNK, D), jnp.float32),
            pltpu.VMEM_SHARED((SP_ROWS, D), jnp.float32),
            pltpu.SemaphoreType.DMA,
            pltpu.SemaphoreType.DMA,
        ],
    )(_sc_scatter_body)


def _sc_scatter_body(y_hbm, src_hbm, dst_hbm, nbb_hbm, out_hbm, src_v, dst_v,
                nb_v, rows0_v, rows1_v, zero_v, acc_sh, sem0, sem1):
    cid = lax.axis_index("c")
    sid = lax.axis_index("s")

    zv = jnp.zeros((16,), jnp.float32)
    lane = lax.iota(jnp.int32, 16)
    z16 = lane * 0

    def zfill(k, carry):
        for c in range(D // 16):
            zero_v[k, 16 * c:16 * (c + 1)] = zv
        return carry

    lax.fori_loop(0, ZCHUNK, zfill, 0)

    zbase = sid * ZROWS

    for r in range(NROUND):
        q = cid + 2 * r

        def zcopy(k, carry):
            pltpu.sync_copy(zero_v, acc_sh.at[pl.ds(zbase + k * ZCHUNK,
                                                    ZCHUNK)])
            return carry

        lax.fori_loop(0, ZROWS // ZCHUNK, zcopy, 0)
        rem = ZROWS % ZCHUNK
        pltpu.sync_copy(zero_v.at[pl.ds(0, rem)],
                        acc_sh.at[pl.ds(zbase + (ZROWS // ZCHUNK) * ZCHUNK,
                                        rem)])
        plsc.subcore_barrier()

        def rel_body(j, carry):
            pltpu.sync_copy(src_hbm.at[j, sid], src_v)
            pltpu.sync_copy(dst_hbm.at[j, sid], dst_v)
            pltpu.sync_copy(nbb_hbm.at[j], nb_v)
            bq = nb_v[sid, pl.ds(q, 16)]
            plo = bq[0]
            phi = bq[1]

            @pl.when(phi > plo)
            def _pipe():
                pltpu.async_copy(y_hbm.at[src_v.at[2 * plo]], rows0_v, sem0)

                def pair_body(h, c2):
                    b0 = 2 * h
                    b1 = 2 * h + 1
                    pltpu.make_async_copy(y_hbm.at[src_v.at[b0]], rows0_v,
                                          sem0).wait()
                    pltpu.async_copy(y_hbm.at[src_v.at[b1]], rows1_v, sem1)
                    pltpu.sync_copy(rows0_v, acc_sh.at[dst_v.at[b0]],
                                    add=True)
                    pltpu.make_async_copy(y_hbm.at[src_v.at[b1]], rows1_v,
                                          sem1).wait()
                    bn = jnp.minimum(b0 + 2, 2 * phi - 1)
                    pltpu.async_copy(y_hbm.at[src_v.at[bn]], rows0_v, sem0)
                    pltpu.sync_copy(rows1_v, acc_sh.at[dst_v.at[b1]],
                                    add=True)
                    return c2

                lax.fori_loop(plo, phi, pair_body, 0)
                pltpu.make_async_copy(y_hbm.at[src_v.at[0]], rows0_v,
                                      sem0).wait()

            return carry

        lax.fori_loop(0, NUM_REL, rel_body, 0)
        plsc.subcore_barrier()

        @pl.when(sid < WB_TILES)
        def _wb_full():
            pltpu.sync_copy(acc_sh.at[pl.ds(zbase, ZROWS)],
                            out_hbm.at[q, pl.ds(zbase, ZROWS)])

        @pl.when(sid == WB_TILES)
        def _wb_part():
            pltpu.sync_copy(acc_sh.at[pl.ds(zbase, WB_REM)],
                            out_hbm.at[q, pl.ds(zbase, WB_REM)])

        if r < NROUND - 1:
            plsc.subcore_barrier()


def _prep_indices(indexes):
    idx = indexes.astype(jnp.int32)
    dst = idx[:, 0::2].T
    src = idx[:, 1::2].T
    pad = EPAD - E
    ar = jnp.arange(pad, dtype=jnp.int32)
    pad_src = (ar * 977) % N
    pad_dst = NPLANE * Q + (ar % 960)
    srcp = jnp.concatenate(
        [src, jnp.broadcast_to(pad_src, (NUM_REL, pad))], axis=1)
    dstp = jnp.concatenate(
        [dst, jnp.broadcast_to(pad_dst, (NUM_REL, pad))], axis=1)
    jrel = jnp.arange(NUM_REL, dtype=jnp.int32)[:, None]
    srcf = ((jrel + 1) * N + srcp).reshape(NUM_REL, N_TILES, SEG)
    dstg = dstp.reshape(NUM_REL, N_TILES, SEG)

    plane = dstg // Q
    rank = jnp.zeros_like(dstg)
    counts = []
    for p in range(NPLANE + 1):
        mp = plane == p
        cump = jnp.cumsum(mp.astype(jnp.int32), axis=-1)
        rank = rank + jnp.where(mp, cump - 1, 0)
        counts.append(cump[..., -1])
    c = jnp.stack(counts, axis=-1)
    prs = (c + 2 * B_E - 1) // (2 * B_E)
    pp_start = jnp.concatenate(
        [jnp.zeros_like(prs[:, :, :1]), jnp.cumsum(prs, axis=-1)],
        axis=-1)
    pbs = jnp.zeros_like(dstg)
    for p in range(NPLANE + 1):
        pbs = pbs + jnp.where(plane == p, pp_start[:, :, p:p + 1], 0)
    pos = pbs * (2 * B_E) + rank
    sl_id = (jnp.arange(NUM_REL * N_TILES, dtype=jnp.int32)
             .reshape(NUM_REL, N_TILES, 1))
    gpos = (sl_id * CAP + pos).reshape(-1)
    ag = jnp.arange(NUM_REL * N_TILES * CAP, dtype=jnp.int32)
    dum_src = (ag * 977) % N
    dum_dst = Q + (ag * 37) % (SP_ROWS - Q - 8)
    out_src = dum_src.at[gpos].set(srcf.reshape(-1), unique_indices=True)
    out_dst = dum_dst.at[gpos].set((dstg - plane * Q).reshape(-1),
                                   unique_indices=True)
    out_src = out_src.reshape(NUM_REL, N_TILES, CAPB, B_E)
    out_dst = out_dst.reshape(NUM_REL, N_TILES, CAPB, B_E)

    nbb = jnp.concatenate(
        [pp_start,
         jnp.zeros((NUM_REL, N_TILES, 32 - pp_start.shape[-1]), jnp.int32)],
        axis=-1)
    return out_src, out_dst, nbb


def kernel(nodes, params, indexes):
    p = params
    src_flat, dst_flat, nbb = _prep_indices(indexes)
    r1 = lambda a: a.reshape(1, D)
    wcats = [
        jnp.transpose(
            jnp.concatenate([p['ctr_w'][i:i + 1], p['rel_w'][i]], axis=0),
            (1, 0, 2)).reshape(D, (NUM_REL + 1) * D)
        for i in range(4)
    ]
    pv = [
        p['input_w1'], r1(p['input_b1']), p['input_w2'],
        r1(p['input_gn_w']), r1(p['input_gn_b']),
        p['seg_w1'], r1(p['seg_b1']), p['seg_w2'],
        r1(p['seg_gn_w']), r1(p['seg_gn_b']),
        p['meta_w'], r1(p['meta_gn_w']), r1(p['meta_gn_b']),
        wcats[0],
    ]
    feat, y = _stage_a(nodes, pv)
    sc_scatter = _make_sc_scatter()
    for i in range(4):
        y_flat = y.reshape((NUM_REL + 1) * N, D)
        temp4 = sc_scatter(y_flat, src_flat, dst_flat, nbb)
        wcat_next = wcats[i + 1] if i < 3 else None
        feat, y = _stage_d(
            temp4, y, feat,
            r1(p['norm_w'][i]), r1(p['norm_b'][i]),
            p['ctr2_w'][i],
            r1(p['ctr2_gn_w'][i]), r1(p['ctr2_gn_b'][i]),
            wcat_next)
    return (feat, nodes[:, :2])


# speedup vs baseline: 9.8755x; 1.8702x over previous
import functools

import jax
import jax.numpy as jnp
from jax import lax
from jax.experimental import pallas as pl
from jax.experimental.pallas import tpu as pltpu
from jax.experimental.pallas import tpu_sc as plsc

N = 50000
E = 40000
D = 128
NUM_REL = 14
NB = 1000
GRID = N // NB
NBD = 1000
GRIDD = N // NBD
B_E = 128
N_TILES = 16
EPAD = 40960
NBATCH = EPAD // (N_TILES * B_E)
NPLANE = 8
NROUND = NPLANE // 2
SEG = EPAD // N_TILES
CAPB = SEG // B_E + 16
CAP = CAPB * B_E
Q = 7000
ZROWS = 456
SP_ROWS = N_TILES * ZROWS
WB_TILES = Q // ZROWS
WB_REM = Q - WB_TILES * ZROWS
ZCHUNK = 128


def _gn_rows(x, w, b):
    mu = jnp.mean(x, axis=1, keepdims=True)
    xc = x - mu
    var = jnp.mean(xc * xc, axis=1, keepdims=True)
    return xc * lax.rsqrt(var + 1e-5) * w + b


def _stage_a_body(nodes, iw1, ib1, iw2, ignw, ignb, sw1, sb1, sw2, sgnw,
                  sgnb, mw, mgnw, mgnb, wcat, feat_out, y_out):
    x = nodes[...]
    f = jnp.maximum(x[:, 0:1] * iw1[0:1, :] + x[:, 1:2] * iw1[1:2, :]
                    + ib1[...], 0.0)
    f = _gn_rows(jnp.dot(f, iw2[...], preferred_element_type=jnp.float32),
                 ignw[...], ignb[...])
    s = jnp.maximum(x[:, 2:3] * sw1[0:1, :] + x[:, 3:4] * sw1[1:2, :]
                    + sb1[...], 0.0)
    s = _gn_rows(jnp.dot(s, sw2[...], preferred_element_type=jnp.float32),
                 sgnw[...], sgnb[...])
    f = jnp.maximum(f + s, 0.0)
    m = jnp.dot(f, mw[0:D, :], preferred_element_type=jnp.float32)
    m = (m + x[:, 4:5] * mw[D:D + 1, :] + x[:, 5:6] * mw[D + 1:D + 2, :]
         + x[:, 6:7] * mw[D + 2:D + 3, :] + x[:, 7:8] * mw[D + 3:D + 4, :])
    f = jnp.maximum(_gn_rows(m, mgnw[...], mgnb[...]), 0.0)
    feat_out[...] = f
    y_big = jnp.dot(f, wcat[...], preferred_element_type=jnp.float32)
    for k in range(NUM_REL + 1):
        y_out[k] = y_big[:, k * D:(k + 1) * D]


def _stage_a(nodes, pv):
    full = lambda shp: pl.BlockSpec(shp, lambda i: tuple(0 for _ in shp))
    return pl.pallas_call(
        _stage_a_body,
        grid=(GRID,),
        in_specs=[
            pl.BlockSpec((NB, 8), lambda i: (i, 0)),
            full((2, D)), full((1, D)), full((D, D)), full((1, D)),
            full((1, D)),
            full((2, D)), full((1, D)), full((D, D)), full((1, D)),
            full((1, D)),
            full((D + 4, D)), full((1, D)), full((1, D)),
            full((D, (NUM_REL + 1) * D)),
        ],
        out_specs=[
            pl.BlockSpec((NB, D), lambda i: (i, 0)),
            pl.BlockSpec((NUM_REL + 1, NB, D), lambda i: (0, i, 0)),
        ],
        out_shape=[
            jax.ShapeDtypeStruct((N, D), jnp.float32),
            jax.ShapeDtypeStruct((NUM_REL + 1, N, D), jnp.float32),
        ],
    )(nodes, *pv)


def _stage_d_feat(t, y, res, nw, nb_, c2w, c2gw, c2gb):
    temp = t[0] + y[0]
    x1 = jnp.maximum(_gn_rows(temp, nw[...], nb_[...]), 0.0)
    z = _gn_rows(jnp.dot(x1, c2w[...], preferred_element_type=jnp.float32),
                 c2gw[...], c2gb[...])
    return jnp.maximum(z + res[...], 0.0)


def _stage_d_body(t, y, res, nw, nb_, c2w, c2gw, c2gb, wcat, feat_out,
                  y_out):
    f = _stage_d_feat(t, y, res, nw, nb_, c2w, c2gw, c2gb)
    feat_out[...] = f
    y_big = jnp.dot(f, wcat[...], preferred_element_type=jnp.float32)
    for k in range(NUM_REL + 1):
        y_out[k] = y_big[:, k * D:(k + 1) * D]


def _stage_d_body_last(t, y, res, nw, nb_, c2w, c2gw, c2gb, feat_out):
    feat_out[...] = _stage_d_feat(t, y, res, nw, nb_, c2w, c2gw, c2gb)


_BPP = Q // NBD


def _stage_d(temp4, y, res, nw, nb_, c2w, c2gw, c2gb, wcat):
    last = wcat is None
    full = lambda shp: pl.BlockSpec(shp, lambda i: tuple(0 for _ in shp))
    body = _stage_d_body_last if last else _stage_d_body
    in_specs = [
        pl.BlockSpec((1, NBD, D), lambda i: (i // _BPP, i % _BPP, 0)),
        pl.BlockSpec((1, NBD, D), lambda i: (0, i, 0)),
        pl.BlockSpec((NBD, D), lambda i: (i, 0)),
        full((1, D)), full((1, D)), full((D, D)), full((1, D)),
        full((1, D)),
    ]
    out_specs = [pl.BlockSpec((NBD, D), lambda i: (i, 0))]
    out_shape = [jax.ShapeDtypeStruct((N, D), jnp.float32)]
    args = [temp4, y, res, nw, nb_, c2w, c2gw, c2gb]
    if not last:
        in_specs.append(full((D, (NUM_REL + 1) * D)))
        out_specs.append(pl.BlockSpec((NUM_REL + 1, NBD, D),
                                      lambda i: (0, i, 0)))
        out_shape.append(
            jax.ShapeDtypeStruct((NUM_REL + 1, N, D), jnp.float32))
        args.append(wcat)
    res_ = pl.pallas_call(
        body, grid=(GRIDD,), in_specs=in_specs, out_specs=out_specs,
        out_shape=out_shape,
    )(*args)
    return res_ if not last else (res_[0], None)


def _make_sc_scatter():
    return functools.partial(
        pl.kernel,
        mesh=plsc.VectorSubcoreMesh(core_axis_name="c",
                                    subcore_axis_name="s", num_cores=2),
        out_type=jax.ShapeDtypeStruct((NPLANE, SP_ROWS, D), jnp.float32),
        scratch_types=[
            pltpu.VMEM((CAPB, B_E), jnp.int32),
            pltpu.VMEM((CAPB, B_E), jnp.int32),
            pltpu.VMEM((N_TILES, 32), jnp.int32),
            pltpu.VMEM((B_E, D), jnp.float32),
            pltpu.VMEM((B_E, D), jnp.float32),
            pltpu.VMEM((ZCHUNK, D), jnp.float32),
            pltpu.VMEM_SHARED((SP_ROWS, D), jnp.float32),
            pltpu.SemaphoreType.DMA,
            pltpu.SemaphoreType.DMA,
        ],
    )(_sc_scatter_body)


def _sc_scatter_body(y_hbm, src_hbm, dst_hbm, nbb_hbm, out_hbm, src_v, dst_v,
                nb_v, rows0_v, rows1_v, zero_v, acc_sh, sem0, sem1):
    cid = lax.axis_index("c")
    sid = lax.axis_index("s")

    zv = jnp.zeros((16,), jnp.float32)
    lane = lax.iota(jnp.int32, 16)
    z16 = lane * 0

    def zfill(k, carry):
        for c in range(D // 16):
            zero_v[k, 16 * c:16 * (c + 1)] = zv
        return carry

    lax.fori_loop(0, ZCHUNK, zfill, 0)

    zbase = sid * ZROWS

    for r in range(NROUND):
        q = cid + 2 * r

        def zcopy(k, carry):
            pltpu.sync_copy(zero_v, acc_sh.at[pl.ds(zbase + k * ZCHUNK,
                                                    ZCHUNK)])
            return carry

        lax.fori_loop(0, ZROWS // ZCHUNK, zcopy, 0)
        rem = ZROWS % ZCHUNK
        pltpu.sync_copy(zero_v.at[pl.ds(0, rem)],
                        acc_sh.at[pl.ds(zbase + (ZROWS // ZCHUNK) * ZCHUNK,
                                        rem)])
        plsc.subcore_barrier()

        def rel_body(j, carry):
            pltpu.sync_copy(src_hbm.at[j, sid], src_v)
            pltpu.sync_copy(dst_hbm.at[j, sid], dst_v)
            pltpu.sync_copy(nbb_hbm.at[j], nb_v)
            bq = nb_v[sid, pl.ds(q, 16)]
            plo = bq[0]
            phi = bq[1]

            @pl.when(phi > plo)
            def _pipe():
                pltpu.async_copy(y_hbm.at[src_v.at[2 * plo]], rows0_v, sem0)

                def pair_body(h, c2):
                    b0 = 2 * h
                    b1 = 2 * h + 1
                    pltpu.make_async_copy(y_hbm.at[src_v.at[b0]], rows0_v,
                                          sem0).wait()
                    pltpu.async_copy(y_hbm.at[src_v.at[b1]], rows1_v, sem1)
                    pltpu.sync_copy(rows0_v, acc_sh.at[dst_v.at[b0]],
                                    add=True)
                    pltpu.make_async_copy(y_hbm.at[src_v.at[b1]], rows1_v,
                                          sem1).wait()
                    bn = jnp.minimum(b0 + 2, 2 * phi - 1)
                    pltpu.async_copy(y_hbm.at[src_v.at[bn]], rows0_v, sem0)
                    pltpu.sync_copy(rows1_v, acc_sh.at[dst_v.at[b1]],
                                    add=True)
                    return c2

                lax.fori_loop(plo, phi, pair_body, 0)
                pltpu.make_async_copy(y_hbm.at[src_v.at[0]], rows0_v,
                                      sem0).wait()

            return carry

        lax.fori_loop(0, NUM_REL, rel_body, 0)
        plsc.subcore_barrier()

        @pl.when(sid < WB_TILES)
        def _wb_full():
            pltpu.sync_copy(acc_sh.at[pl.ds(zbase, ZROWS)],
                            out_hbm.at[q, pl.ds(zbase, ZROWS)])

        @pl.when(sid == WB_TILES)
        def _wb_part():
            pltpu.sync_copy(acc_sh.at[pl.ds(zbase, WB_REM)],
                            out_hbm.at[q, pl.ds(zbase, WB_REM)])

        if r < NROUND - 1:
            plsc.subcore_barrier()


def _prep_indices(indexes):
    idx = indexes.astype(jnp.int32)
    dst = idx[:, 0::2].T
    src = idx[:, 1::2].T
    pad = EPAD - E
    ar = jnp.arange(pad, dtype=jnp.int32)
    pad_src = (ar * 977) % N
    pad_dst = NPLANE * Q + (ar % 960)
    srcp = jnp.concatenate(
        [src, jnp.broadcast_to(pad_src, (NUM_REL, pad))], axis=1)
    dstp = jnp.concatenate(
        [dst, jnp.broadcast_to(pad_dst, (NUM_REL, pad))], axis=1)
    jrel = jnp.arange(NUM_REL, dtype=jnp.int32)[:, None]
    srcf = ((jrel + 1) * N + srcp).reshape(NUM_REL, N_TILES, SEG)
    dstg = dstp.reshape(NUM_REL, N_TILES, SEG)

    plane = dstg // Q
    rank = jnp.zeros_like(dstg)
    counts = []
    for p in range(NPLANE + 1):
        mp = plane == p
        cump = jnp.cumsum(mp.astype(jnp.int32), axis=-1)
        rank = rank + jnp.where(mp, cump - 1, 0)
        counts.append(cump[..., -1])
    c = jnp.stack(counts, axis=-1)
    prs = (c + 2 * B_E - 1) // (2 * B_E)
    pp_start = jnp.concatenate(
        [jnp.zeros_like(prs[:, :, :1]), jnp.cumsum(prs, axis=-1)],
        axis=-1)
    pbs = jnp.zeros_like(dstg)
    for p in range(NPLANE + 1):
        pbs = pbs + jnp.where(plane == p, pp_start[:, :, p:p + 1], 0)
    pos = pbs * (2 * B_E) + rank
    sl_id = (jnp.arange(NUM_REL * N_TILES, dtype=jnp.int32)
             .reshape(NUM_REL, N_TILES, 1))
    gpos = (sl_id * CAP + pos).reshape(-1)
    src_node = srcp.reshape(NUM_REL, N_TILES, SEG)
    dst_loc = dstg - plane * Q
    packed = (dst_loc << 16) | src_node

    def _dum(ix):
        return (((ix * 37) % (SP_ROWS - Q - 8) + Q) << 16) | ((ix * 977) % N)

    ag = jnp.arange(NUM_REL * N_TILES * CAP, dtype=jnp.int32)
    out_p = _dum(ag).at[gpos].add(packed.reshape(-1) - _dum(gpos))
    out_p = out_p.reshape(NUM_REL, N_TILES, CAP)
    out_src = (jrel[:, :, None] + 1) * N + (out_p & 0xFFFF)
    out_dst = out_p >> 16
    out_src = out_src.reshape(NUM_REL, N_TILES, CAPB, B_E)
    out_dst = out_dst.reshape(NUM_REL, N_TILES, CAPB, B_E)

    nbb = jnp.concatenate(
        [pp_start,
         jnp.zeros((NUM_REL, N_TILES, 32 - pp_start.shape[-1]), jnp.int32)],
        axis=-1)
    return out_src, out_dst, nbb


def kernel(nodes, params, indexes):
    p = params
    src_flat, dst_flat, nbb = _prep_indices(indexes)
    r1 = lambda a: a.reshape(1, D)
    wcats = [
        jnp.transpose(
            jnp.concatenate([p['ctr_w'][i:i + 1], p['rel_w'][i]], axis=0),
            (1, 0, 2)).reshape(D, (NUM_REL + 1) * D)
        for i in range(4)
    ]
    pv = [
        p['input_w1'], r1(p['input_b1']), p['input_w2'],
        r1(p['input_gn_w']), r1(p['input_gn_b']),
        p['seg_w1'], r1(p['seg_b1']), p['seg_w2'],
        r1(p['seg_gn_w']), r1(p['seg_gn_b']),
        p['meta_w'], r1(p['meta_gn_w']), r1(p['meta_gn_b']),
        wcats[0],
    ]
    feat, y = _stage_a(nodes, pv)
    sc_scatter = _make_sc_scatter()
    for i in range(4):
        y_flat = y.reshape((NUM_REL + 1) * N, D)
        temp4 = sc_scatter(y_flat, src_flat, dst_flat, nbb)
        wcat_next = wcats[i + 1] if i < 3 else None
        feat, y = _stage_d(
            temp4, y, feat,
            r1(p['norm_w'][i]), r1(p['norm_b'][i]),
            p['ctr2_w'][i],
            r1(p['ctr2_gn_w'][i]), r1(p['ctr2_gn_b'][i]),
            wcat_next)
    return (feat, nodes[:, :2])
